# baseline (speedup 1.0000x reference)
import functools

import jax
import jax.numpy as jnp
from jax import lax
from jax.experimental import pallas as pl
from jax.experimental.pallas import tpu as pltpu
from jax.experimental.pallas import tpu_sc as plsc

NS_TOT = 1600000
OUT = 100000
PAD = 100352
NC, NSUB, L = 2, 16, 16
NW = NC * NSUB
ROWS = NS_TOT // 128
RPT = ROWS // NW
EXTRA = ROWS - RPT * NW
FULL_CHUNKS = 24
TAIL_HI = RPT + 1 - FULL_CHUNKS * 16
TAIL_LO = RPT - FULL_CHUNKS * 16
SLICE = PAD // NSUB
EPS = 1e-7


def _sc_partials(att, idx):
    mesh = plsc.VectorSubcoreMesh(core_axis_name="c", subcore_axis_name="s")

    @functools.partial(
        pl.kernel,
        out_type=jax.ShapeDtypeStruct((NC, PAD), jnp.float32),
        mesh=mesh,
        compiler_params=pltpu.CompilerParams(
            needs_layout_passes=False, use_tc_tiling_on_sc=False),
        scratch_types=[
            pltpu.VMEM((16, 128), jnp.int32),
            pltpu.VMEM((16, 128), jnp.float32),
            pltpu.VMEM((SLICE,), jnp.float32),
            pltpu.VMEM_SHARED((PAD,), jnp.float32),
            pltpu.SemaphoreType.DMA,
            pltpu.SemaphoreType.DMA,
        ],
    )
    def k(att_hbm, idx_hbm, out_hbm, ib, vb, zb, shared, sem_in, sem_sc):
        c = lax.axis_index("c")
        s = lax.axis_index("s")
        wid = c * NSUB + s

        def zg(g, _):
            zb[pl.ds(g * L, L)] = jnp.zeros((L,), jnp.float32)
            return _
        lax.fori_loop(0, SLICE // L, zg, None)
        off = pl.multiple_of(s * SLICE, 8)
        pltpu.sync_copy(zb, shared.at[pl.ds(off, SLICE)])
        plsc.subcore_barrier()

        base_row = wid * RPT + jnp.minimum(wid, EXTRA)

        def do_chunk(row0, nrows):
            rsl = pl.ds(row0, nrows)
            dsl = pl.ds(0, nrows)
            cp1 = pltpu.async_copy(idx_hbm.at[rsl], ib.at[dsl], sem_in)
            cp2 = pltpu.async_copy(att_hbm.at[rsl], vb.at[dsl], sem_in)
            cp1.wait()
            cp2.wait()

            cps = [
                pltpu.async_copy(
                    vb.at[jj], shared.at[ib.at[jj]], sem_sc, add=True)
                for jj in range(nrows)
            ]
            for cp in cps:
                cp.wait()

        def chunk_loop(kk, _):
            do_chunk(base_row + kk * 16, 16)
            return _
        lax.fori_loop(0, FULL_CHUNKS, chunk_loop, None)

        tail_row = base_row + FULL_CHUNKS * 16

        @pl.when(wid < EXTRA)
        def _():
            do_chunk(tail_row, TAIL_HI)

        @pl.when(wid >= EXTRA)
        def _():
            do_chunk(tail_row, TAIL_LO)

        plsc.subcore_barrier()
        pltpu.sync_copy(shared.at[pl.ds(off, SLICE)],
                        out_hbm.at[c, pl.ds(off, SLICE)])

    return k(att, idx)


def _tc_merge(p):
    def body(p_ref, o_ref):
        o_ref[...] = p_ref[0] + p_ref[1]

    return pl.pallas_call(
        body,
        out_shape=jax.ShapeDtypeStruct((PAD // 128, 128), jnp.float32),
    )(p)


def kernel(weights, ray_indices, num_rays, distances):
    att = (weights / (distances + jnp.float32(EPS))).reshape(ROWS, 128)
    idx = ray_indices.reshape(ROWS, 128)
    partials = _sc_partials(att, idx)
    merged = _tc_merge(partials.reshape(NC, PAD // 128, 128))
    return merged.reshape(PAD)[:OUT][:, None]
